# baseline (speedup 1.0000x reference)
import functools

import jax
import jax.numpy as jnp
import numpy as np
from jax import lax
from jax.experimental import pallas as pl
from jax.experimental.pallas import tpu as pltpu
from jax.experimental.pallas import tpu_sc as plsc

KNN = 16
BN_SC = float(1.0 / np.sqrt(1.0 + 1e-5))
POSR = 1.0
PPAD = 16

_SC_NC = 2
_SC_NS = 16
_NW = _SC_NC * _SC_NS


@functools.lru_cache(maxsize=None)
def _sc_gather_call(V, D, E_pad, chunk, steps):
    mesh = plsc.VectorSubcoreMesh(core_axis_name="c", subcore_axis_name="s")

    def body(table_hbm, idx_hbm, out_hbm, idx_v, rows_v, sem):
        wid = lax.axis_index("s") * _SC_NC + lax.axis_index("c")
        base = wid * (chunk * steps)

        def step(t, carry):
            off = base + t * chunk
            pltpu.sync_copy(idx_hbm.at[pl.ds(off, chunk)], idx_v)
            pltpu.async_copy(table_hbm.at[idx_v], rows_v, sem).wait()
            pltpu.sync_copy(rows_v, out_hbm.at[pl.ds(off, chunk)])
            return carry

        lax.fori_loop(0, steps, step, 0)

    return pl.kernel(
        body,
        out_type=jax.ShapeDtypeStruct((E_pad, D), jnp.float32),
        mesh=mesh,
        scratch_types=[
            pltpu.VMEM((chunk,), jnp.int32),
            pltpu.VMEM((chunk, D), jnp.float32),
            pltpu.SemaphoreType.DMA,
        ],
        compiler_params=pltpu.CompilerParams(use_tc_tiling_on_sc=False),
    )


def _gather_rows(table, idx):
    V, D = table.shape
    E = idx.shape[0]
    assert D % 16 == 0
    chunk = min(128, (120000 // (D + 1)) // 8 * 8)
    steps = -(-E // (_NW * chunk))
    E_pad = _NW * chunk * steps
    if E_pad != E:
        idx = jnp.concatenate([idx, jnp.zeros((E_pad - E,), jnp.int32)])
    out = _sc_gather_call(V, D, E_pad, chunk, steps)(table, idx)
    return out[:E] if E_pad != E else out


def _knn_body(posq_ref, post_ref, out_ref, *, self_mask, tq, p):
    j = pl.program_id(1)
    ch = min(512, p)
    nch = p // ch
    qx = posq_ref[0, :, 0:1]
    qy = posq_ref[0, :, 1:2]
    qz = posq_ref[0, :, 2:3]
    lane = lax.broadcasted_iota(jnp.int32, (tq, ch), 1)
    row = j * tq + lax.broadcasted_iota(jnp.int32, (tq, 1), 0)
    rv = ri = None
    for c in range(nch):
        sx = post_ref[0, 0:1, c * ch : (c + 1) * ch]
        sy = post_ref[0, 1:2, c * ch : (c + 1) * ch]
        sz = post_ref[0, 2:3, c * ch : (c + 1) * ch]
        dx = qx - sx
        dy = qy - sy
        dz = qz - sz
        d = dx * dx + dy * dy + dz * dz
        if self_mask:
            d = jnp.where(lane + c * ch == row, d + 1e10, d)
        vals, gidx = [], []
        for k in range(KNN):
            m = jnp.min(d, axis=1, keepdims=True)
            li = jnp.min(jnp.where(d == m, lane, jnp.int32(1 << 30)), axis=1, keepdims=True)
            vals.append(m)
            gidx.append(li + c * ch)
            d = jnp.where(lane == li, jnp.float32(3e38), d)
        cv = jnp.concatenate(vals, axis=1)
        ci = jnp.concatenate(gidx, axis=1)
        if rv is None:
            rv, ri = cv, ci
        else:
            av = jnp.concatenate([rv, cv], axis=1)
            ai = jnp.concatenate([ri, ci], axis=1)
            nv, ni = [], []
            for k in range(KNN):
                m = jnp.min(av, axis=1, keepdims=True)
                mi = jnp.min(jnp.where(av == m, ai, jnp.int32(1 << 30)), axis=1,
                             keepdims=True)
                nv.append(m)
                ni.append(mi)
                av = jnp.where((av == m) & (ai == mi), jnp.float32(3e38), av)
            rv = jnp.concatenate(nv, axis=1)
            ri = jnp.concatenate(ni, axis=1)
    out_ref[0] = ri


def _knn(pos_q, pos_src_t, self_mask):
    B, Pq, _ = pos_q.shape
    P = pos_src_t.shape[2]
    tq = min(128, Pq)
    grid = (B, Pq // tq)
    return pl.pallas_call(
        functools.partial(_knn_body, self_mask=self_mask, tq=tq, p=P),
        grid=grid,
        in_specs=[
            pl.BlockSpec((1, tq, PPAD), lambda b, j: (b, j, 0)),
            pl.BlockSpec((1, 8, P), lambda b, j: (b, 0, 0)),
        ],
        out_specs=pl.BlockSpec((1, tq, KNN), lambda b, j: (b, j, 0)),
        out_shape=jax.ShapeDtypeStruct((B, Pq, KNN), jnp.int32),
    )(pos_q, pos_src_t)


def _fps_body(psq_ref, out_ref, *, n, p, rows, nb):
    lane = (lax.broadcasted_iota(jnp.int32, (rows, 128), 0) * 128
            + lax.broadcasted_iota(jnp.int32, (rows, 128), 1))
    coords = []
    for b in range(nb):
        coords.append((psq_ref[b, 0:rows, :],
                       psq_ref[b, rows : 2 * rows, :],
                       psq_ref[b, 2 * rows :, :]))

    def red2(x, op):
        return op(op(x, axis=0, keepdims=True), axis=1, keepdims=True)

    def dist(b, sx, sy, sz):
        X, Y, Z = coords[b]
        dx = X - sx
        dy = Y - sy
        dz = Z - sz
        return dx * dx + dy * dy + dz * dz

    def extract(b, oh):
        X, Y, Z = coords[b]
        return (red2(jnp.where(oh, X, 0.0), jnp.sum),
                red2(jnp.where(oh, Y, 0.0), jnp.sum),
                red2(jnp.where(oh, Z, 0.0), jnp.sum))

    carry0 = []
    oh0 = lane == 0
    for b in range(nb):
        sx, sy, sz = extract(b, oh0)
        mind = dist(b, sx, sy, sz)
        mind = jnp.where(lane < p, mind, -1.0)
        out_ref[b, 0, 0] = jnp.int32(0)
        carry0.append(mind)
        carry0.append(jnp.zeros((1, 1), jnp.int32))

    def step(i, carry):
        nxts = []
        for b in range(nb):
            mind, prev = carry[2 * b], carry[2 * b + 1]
            sx, sy, sz = extract(b, lane == prev)
            mind = jnp.minimum(mind, dist(b, sx, sy, sz))
            m = red2(mind, jnp.max)
            nxt = red2(jnp.where(mind == m, lane, p), jnp.min)
            out_ref[b, 0, i] = nxt[0, 0]
            nxts.append((mind, nxt))
        return tuple(x for mn in nxts for x in mn)

    lax.fori_loop(1, n, step, tuple(carry0))


def _fps(pos_sq, n, p):
    B, r3, _ = pos_sq.shape
    rows = r3 // 3
    out = pl.pallas_call(
        functools.partial(_fps_body, n=n, p=p, rows=rows, nb=B),
        in_specs=[pl.BlockSpec((B, r3, 128), lambda: (0, 0, 0))],
        out_specs=pl.BlockSpec(
            (B, 1, n), lambda: (0, 0, 0), memory_space=pltpu.SMEM
        ),
        out_shape=jax.ShapeDtypeStruct((B, 1, n), jnp.int32),
    )(pos_sq)
    return out[:, 0, :]


def _dense_body(x_ref, w_ref, b_ref, o_ref, *, scale):
    y = jnp.dot(x_ref[0], w_ref[...], preferred_element_type=jnp.float32)
    y = scale * (y + b_ref[...][None, :])
    o_ref[0] = jnp.maximum(y, 0.0)


def _dense(x, w, b, scale, tp):
    B, P, Cin = x.shape
    Cout = w.shape[1]
    grid = (B, P // tp)
    return pl.pallas_call(
        functools.partial(_dense_body, scale=scale),
        grid=grid,
        in_specs=[
            pl.BlockSpec((1, tp, Cin), lambda b, j: (b, j, 0)),
            pl.BlockSpec((Cin, Cout), lambda b, j: (0, 0)),
            pl.BlockSpec((Cout,), lambda b, j: (0,)),
        ],
        out_specs=pl.BlockSpec((1, tp, Cout), lambda b, j: (b, j, 0)),
        out_shape=jax.ShapeDtypeStruct((B, P, Cout), jnp.float32),
    )(x, w, b)


def _latent_body(x_ref, w_ref, b_ref, o_ref, *, scale):
    y = jnp.dot(x_ref[...], w_ref[...], preferred_element_type=jnp.float32)
    y = scale * (y + b_ref[...][None, :])
    o_ref[...] = jnp.maximum(y, 0.0)


def _latent(x, w, b, scale, tc):
    R, Cin = x.shape
    Cout = w.shape[1]
    grid = (Cout // tc,)
    return pl.pallas_call(
        functools.partial(_latent_body, scale=scale),
        grid=grid,
        in_specs=[
            pl.BlockSpec((R, Cin), lambda j: (0, 0)),
            pl.BlockSpec((Cin, tc), lambda j: (0, j)),
            pl.BlockSpec((tc,), lambda j: (j,)),
        ],
        out_specs=pl.BlockSpec((R, tc), lambda j: (0, j)),
        out_shape=jax.ShapeDtypeStruct((R, Cout), jnp.float32),
    )(x, w, b)


def _pre_body(g_ref, pos_ref, wi_ref, bi_ref, wl_ref, ws_ref, wd_ref,
              vsp_ref, q_ref, *, c):
    x = g_ref[0, 0]
    for k in range(1, KNN):
        x = jnp.maximum(x, g_ref[k, 0])
    h = jnp.dot(x, wi_ref[...], preferred_element_type=jnp.float32)
    h = jnp.maximum(h + bi_ref[...][None, :], 0.0)
    vsp_ref[0, :, 0:c] = jnp.dot(h, wl_ref[...], preferred_element_type=jnp.float32)
    vsp_ref[0, :, c : 2 * c] = jnp.dot(h, ws_ref[...], preferred_element_type=jnp.float32)
    vsp_ref[0, :, 2 * c : 2 * c + PPAD] = pos_ref[0]
    q_ref[0] = jnp.dot(h, wd_ref[...], preferred_element_type=jnp.float32)


def _pre(g, pos, tb, tp):
    _, B, P, C = g.shape
    D = 2 * C + PPAD
    grid = (B, P // tp)
    return pl.pallas_call(
        functools.partial(_pre_body, c=C),
        grid=grid,
        in_specs=[
            pl.BlockSpec((KNN, 1, tp, C), lambda b, j: (0, b, j, 0)),
            pl.BlockSpec((1, tp, PPAD), lambda b, j: (b, j, 0)),
            pl.BlockSpec((C, C), lambda b, j: (0, 0)),
            pl.BlockSpec((C,), lambda b, j: (0,)),
            pl.BlockSpec((C, C), lambda b, j: (0, 0)),
            pl.BlockSpec((C, C), lambda b, j: (0, 0)),
            pl.BlockSpec((C, C), lambda b, j: (0, 0)),
        ],
        out_specs=[
            pl.BlockSpec((1, tp, D), lambda b, j: (b, j, 0)),
            pl.BlockSpec((1, tp, C), lambda b, j: (b, j, 0)),
        ],
        out_shape=[
            jax.ShapeDtypeStruct((B, P, D), jnp.float32),
            jax.ShapeDtypeStruct((B, P, C), jnp.float32),
        ],
    )(g, pos, tb["lin_in"]["W"], tb["lin_in"]["b"], tb["lin"],
      tb["lin_src"], tb["lin_dst"])


def _mlp2_tile(x, w1, b1, w2, b2):
    t = jnp.dot(x, w1, preferred_element_type=jnp.float32)
    t = jnp.maximum(t + b1[None, :], 0.0)
    t = jnp.dot(t, w2, preferred_element_type=jnp.float32)
    return jnp.maximum(t + b2[None, :], 0.0)


def _edge_body(pos_ref, q_ref, g_ref, pw1_ref, pb1_ref, pw2_ref, pb2_ref,
               aw1_ref, ab1_ref, aw2_ref, ab2_ref, wo_ref, bo_ref, o_ref,
               *, c, tp):
    pos_i = pos_ref[0]
    q = q_ref[0]
    pw1 = pw1_ref[...]
    pb1 = pb1_ref[...]
    pw2 = pw2_ref[...]
    pb2 = pb2_ref[...]
    aw1 = aw1_ref[...]
    ab1 = ab1_ref[...]
    aw2 = aw2_ref[...]
    ab2 = ab2_ref[...]
    m = jnp.full((tp, c), -jnp.inf, jnp.float32)
    l = jnp.zeros((tp, c), jnp.float32)
    acc = jnp.zeros((tp, c), jnp.float32)
    for k in range(KNN + 1):
        gk = g_ref[k, 0]
        vk = gk[:, 0:c]
        sk = gk[:, c : 2 * c]
        pk = gk[:, 2 * c : 2 * c + PPAD]
        rel = pos_i - pk
        delta = _mlp2_tile(rel, pw1, pb1, pw2, pb2)
        alpha = _mlp2_tile(q - sk + delta, aw1, ab1, aw2, ab2)
        mk = jnp.maximum(m, alpha)
        corr = jnp.exp(m - mk)
        pexp = jnp.exp(alpha - mk)
        acc = acc * corr + pexp * (vk + delta)
        l = l * corr + pexp
        m = mk
    out = acc / l
    out = jnp.dot(out, wo_ref[...], preferred_element_type=jnp.float32)
    o_ref[0] = jnp.maximum(out + bo_ref[...][None, :], 0.0)


def _edge(pos, q, g, tb, pw1p, tp):
    _, B, P, D = g.shape
    C = (D - PPAD) // 2
    grid = (B, P // tp)
    return pl.pallas_call(
        functools.partial(_edge_body, c=C, tp=tp),
        grid=grid,
        in_specs=[
            pl.BlockSpec((1, tp, PPAD), lambda b, j: (b, j, 0)),
            pl.BlockSpec((1, tp, C), lambda b, j: (b, j, 0)),
            pl.BlockSpec((KNN + 1, 1, tp, D), lambda b, j: (0, b, j, 0)),
            pl.BlockSpec((PPAD, 64), lambda b, j: (0, 0)),
            pl.BlockSpec((64,), lambda b, j: (0,)),
            pl.BlockSpec((64, C), lambda b, j: (0, 0)),
            pl.BlockSpec((C,), lambda b, j: (0,)),
            pl.BlockSpec((C, 64), lambda b, j: (0, 0)),
            pl.BlockSpec((64,), lambda b, j: (0,)),
            pl.BlockSpec((64, C), lambda b, j: (0, 0)),
            pl.BlockSpec((C,), lambda b, j: (0,)),
            pl.BlockSpec((C, C), lambda b, j: (0, 0)),
            pl.BlockSpec((C,), lambda b, j: (0,)),
        ],
        out_specs=pl.BlockSpec((1, tp, C), lambda b, j: (b, j, 0)),
        out_shape=jax.ShapeDtypeStruct((B, P, C), jnp.float32),
    )(pos, q, g, pw1p, tb["pos_nn"][0]["b"], tb["pos_nn"][1]["W"],
      tb["pos_nn"][1]["b"], tb["attn_nn"][0]["W"], tb["attn_nn"][0]["b"],
      tb["attn_nn"][1]["W"], tb["attn_nn"][1]["b"], tb["lin_out"]["W"],
      tb["lin_out"]["b"])


def _edge0_body(pos_ref, g_ref, w1_ref, b1_ref, wi_ref, bi_ref, wl_ref,
                ws_ref, wd_ref, pw1_ref, pb1_ref, pw2_ref, pb2_ref,
                aw1_ref, ab1_ref, aw2_ref, ab2_ref, wo_ref, bo_ref, o_ref,
                *, c, tp):
    x0 = jnp.maximum(BN_SC * (w1_ref[...] + b1_ref[...][None, :]), 0.0)
    h = jnp.dot(x0, wi_ref[...], preferred_element_type=jnp.float32)
    h = jnp.maximum(h + bi_ref[...][None, :], 0.0)
    vc = jnp.dot(h, wl_ref[...], preferred_element_type=jnp.float32)
    sc = jnp.dot(h, ws_ref[...], preferred_element_type=jnp.float32)
    qc = jnp.dot(h, wd_ref[...], preferred_element_type=jnp.float32)
    qs = qc - sc
    pos_i = pos_ref[0]
    pw1 = pw1_ref[...]
    pb1 = pb1_ref[...]
    pw2 = pw2_ref[...]
    pb2 = pb2_ref[...]
    aw1 = aw1_ref[...]
    ab1 = ab1_ref[...]
    aw2 = aw2_ref[...]
    ab2 = ab2_ref[...]
    m = jnp.full((tp, c), -jnp.inf, jnp.float32)
    l = jnp.zeros((tp, c), jnp.float32)
    acc = jnp.zeros((tp, c), jnp.float32)
    for k in range(KNN + 1):
        pk = g_ref[k, 0]
        rel = pos_i - pk
        delta = _mlp2_tile(rel, pw1, pb1, pw2, pb2)
        alpha = _mlp2_tile(qs + delta, aw1, ab1, aw2, ab2)
        mk = jnp.maximum(m, alpha)
        corr = jnp.exp(m - mk)
        pexp = jnp.exp(alpha - mk)
        acc = acc * corr + pexp * (vc + delta)
        l = l * corr + pexp
        m = mk
    out = acc / l
    out = jnp.dot(out, wo_ref[...], preferred_element_type=jnp.float32)
    o_ref[0] = jnp.maximum(out + bo_ref[...][None, :], 0.0)


def _edge0(pos, g, mlp_in, tb, pw1p, tp):
    B, P, _ = pos.shape
    C = tb["lin"].shape[0]
    grid = (B, P // tp)
    return pl.pallas_call(
        functools.partial(_edge0_body, c=C, tp=tp),
        grid=grid,
        in_specs=[
            pl.BlockSpec((1, tp, PPAD), lambda b, j: (b, j, 0)),
            pl.BlockSpec((KNN + 1, 1, tp, PPAD), lambda b, j: (0, b, j, 0)),
            pl.BlockSpec((1, C), lambda b, j: (0, 0)),
            pl.BlockSpec((C,), lambda b, j: (0,)),
            pl.BlockSpec((C, C), lambda b, j: (0, 0)),
            pl.BlockSpec((C,), lambda b, j: (0,)),
            pl.BlockSpec((C, C), lambda b, j: (0, 0)),
            pl.BlockSpec((C, C), lambda b, j: (0, 0)),
            pl.BlockSpec((C, C), lambda b, j: (0, 0)),
            pl.BlockSpec((PPAD, 64), lambda b, j: (0, 0)),
            pl.BlockSpec((64,), lambda b, j: (0,)),
            pl.BlockSpec((64, C), lambda b, j: (0, 0)),
            pl.BlockSpec((C,), lambda b, j: (0,)),
            pl.BlockSpec((C, 64), lambda b, j: (0, 0)),
            pl.BlockSpec((64,), lambda b, j: (0,)),
            pl.BlockSpec((64, C), lambda b, j: (0, 0)),
            pl.BlockSpec((C,), lambda b, j: (0,)),
            pl.BlockSpec((C, C), lambda b, j: (0, 0)),
            pl.BlockSpec((C,), lambda b, j: (0,)),
        ],
        out_specs=pl.BlockSpec((1, tp, C), lambda b, j: (b, j, 0)),
        out_shape=jax.ShapeDtypeStruct((B, P, C), jnp.float32),
    )(pos, g, mlp_in["W"], mlp_in["b"], tb["lin_in"]["W"], tb["lin_in"]["b"],
      tb["lin"], tb["lin_src"], tb["lin_dst"], pw1p, tb["pos_nn"][0]["b"],
      tb["pos_nn"][1]["W"], tb["pos_nn"][1]["b"], tb["attn_nn"][0]["W"],
      tb["attn_nn"][0]["b"], tb["attn_nn"][1]["W"], tb["attn_nn"][1]["b"],
      tb["lin_out"]["W"], tb["lin_out"]["b"])


def _heads_body(f_ref, wo_ref, bo_ref, wc_ref, bc_ref, off_ref, cls_ref):
    f = f_ref[0]
    off = jnp.dot(f, wo_ref[...], preferred_element_type=jnp.float32)
    off_ref[0] = (off + bo_ref[...][None, :]) * POSR
    cl = jnp.dot(f, wc_ref[...], preferred_element_type=jnp.float32)
    cls_ref[0] = jax.nn.sigmoid(cl + bc_ref[...][None, :])


def _heads(feat, wo, bo, wc, bc, tp):
    B, P, C = feat.shape
    grid = (B, P // tp)
    return pl.pallas_call(
        _heads_body,
        grid=grid,
        in_specs=[
            pl.BlockSpec((1, tp, C), lambda b, j: (b, j, 0)),
            pl.BlockSpec((C, 3), lambda b, j: (0, 0)),
            pl.BlockSpec((3,), lambda b, j: (0,)),
            pl.BlockSpec((C, 1), lambda b, j: (0, 0)),
            pl.BlockSpec((1,), lambda b, j: (0,)),
        ],
        out_specs=[
            pl.BlockSpec((1, tp, 3), lambda b, j: (b, j, 0)),
            pl.BlockSpec((1, tp, 1), lambda b, j: (b, j, 0)),
        ],
        out_shape=[
            jax.ShapeDtypeStruct((B, P, 3), jnp.float32),
            jax.ShapeDtypeStruct((B, P, 1), jnp.float32),
        ],
    )(feat, wo, bo, wc, bc)


def _pad16(w3):
    pad = [(0, 0)] * w3.ndim
    for ax, sz in enumerate(w3.shape):
        if sz == 3:
            pad[ax] = (0, PPAD - 3)
    return jnp.pad(w3, pad)


def _edge_gather(vsp, nbr, B, P):
    D = vsp.shape[-1]
    self_idx = jnp.broadcast_to(
        jnp.arange(P, dtype=nbr.dtype)[None, :, None], (B, P, 1))
    nb = jnp.concatenate([nbr, self_idx], axis=2)
    boff = (jnp.arange(B, dtype=jnp.int32) * P)[:, None, None]
    flat = jnp.transpose(nb + boff, (2, 0, 1)).reshape(-1)
    rows = _gather_rows(vsp.reshape(B * P, D), flat)
    return rows.reshape(KNN + 1, B, P, D)


def _assign_gather(h, assign, B, P, Psub):
    C = h.shape[-1]
    boff = (jnp.arange(B, dtype=jnp.int32) * P)[:, None, None]
    flat = jnp.transpose(assign + boff, (2, 0, 1)).reshape(-1)
    rows = _gather_rows(h.reshape(B * P, C), flat)
    return rows.reshape(KNN, B, Psub, C)


def _pos_t(pos16, B, P):
    return jnp.transpose(pos16[:, :, :8], (0, 2, 1))


def _pos_sq(pos16, B, P):
    rows = -(-P // 128)
    c = jnp.transpose(pos16[:, :, :3], (0, 2, 1))
    c = jnp.pad(c, ((0, 0), (0, 0), (0, rows * 128 - P)))
    return c.reshape(B, 3 * rows, 128)


def kernel(points, vectors, params):
    del vectors
    B, P0, _ = points.shape
    dims = [params["tb"][i]["lin"].shape[0] for i in range(5)]

    tp_lvl = [512, 256, 256, 64, 16]
    ps = [P0]
    for _ in range(4):
        ps.append(int(np.ceil(0.25 * ps[-1])))

    pos16 = _pad16(points)

    post = _pos_t(pos16, B, P0)
    nbr0 = _knn(pos16, post, self_mask=True)
    g0 = _edge_gather(pos16, nbr0, B, P0)
    tb0 = params["tb"][0]
    pw1p0 = _pad16(tb0["pos_nn"][0]["W"])
    x = _edge0(pos16, g0, params["mlp_input"], tb0, pw1p0, tp_lvl[0])

    pos = pos16
    for i in range(4):
        P, Psub, C2 = ps[i], ps[i + 1], dims[i + 1]
        post = _pos_t(pos, B, P)
        fidx = _fps(_pos_sq(pos, B, P), Psub, P)
        boff = (jnp.arange(B, dtype=jnp.int32) * P)[:, None]
        sub_pos = _gather_rows(
            pos.reshape(B * P, PPAD), (fidx + boff).reshape(-1)
        ).reshape(B, Psub, PPAD)
        assign = _knn(sub_pos, post, self_mask=False)
        subt = _pos_t(sub_pos, B, Psub)
        nbr = _knn(sub_pos, subt, self_mask=True)

        td = params["td"][i]
        h = _dense(x, td["W"], td["b"], BN_SC, tp_lvl[i])
        g = _assign_gather(h, assign, B, P, Psub)

        tb = params["tb"][i + 1]
        vsp, q = _pre(g, sub_pos, tb, tp_lvl[i + 1])
        gj = _edge_gather(vsp, nbr, B, Psub)
        pw1p = _pad16(tb["pos_nn"][0]["W"])
        x = _edge(sub_pos, q, gj, tb, pw1p, tp_lvl[i + 1])
        pos = sub_pos

    lat = _latent(x.reshape(B * ps[4], dims[4]), params["lin"]["W"],
                  params["lin"]["b"], BN_SC, 2048)
    feat = lat.reshape(B, -1, 128)
    off, cls = _heads(
        feat, params["offset_fc"]["W"], params["offset_fc"]["b"],
        params["cls_fc"]["W"], params["cls_fc"]["b"], 512)
    return feat, cls[..., 0], off


# speedup vs baseline: 2.2258x; 2.2258x over previous
import functools

import jax
import jax.numpy as jnp
import numpy as np
from jax import lax
from jax.experimental import pallas as pl
from jax.experimental.pallas import tpu as pltpu
from jax.experimental.pallas import tpu_sc as plsc

KNN = 16
BN_SC = float(1.0 / np.sqrt(1.0 + 1e-5))
POSR = 1.0
PPAD = 16

_SC_NC = 2
_SC_NS = 16
_NW = _SC_NC * _SC_NS


@functools.lru_cache(maxsize=None)
def _sc_gather_call(V, D, E_pad, chunk, steps):
    mesh = plsc.VectorSubcoreMesh(core_axis_name="c", subcore_axis_name="s")

    def body(table_hbm, idx_hbm, out_hbm, idx_v, rows_v, sem):
        wid = lax.axis_index("s") * _SC_NC + lax.axis_index("c")
        base = wid * (chunk * steps)

        def step(t, carry):
            off = base + t * chunk
            pltpu.sync_copy(idx_hbm.at[pl.ds(off, chunk)], idx_v)
            pltpu.async_copy(table_hbm.at[idx_v], rows_v, sem).wait()
            pltpu.sync_copy(rows_v, out_hbm.at[pl.ds(off, chunk)])
            return carry

        lax.fori_loop(0, steps, step, 0)

    return pl.kernel(
        body,
        out_type=jax.ShapeDtypeStruct((E_pad, D), jnp.float32),
        mesh=mesh,
        scratch_types=[
            pltpu.VMEM((chunk,), jnp.int32),
            pltpu.VMEM((chunk, D), jnp.float32),
            pltpu.SemaphoreType.DMA,
        ],
        compiler_params=pltpu.CompilerParams(use_tc_tiling_on_sc=False),
    )


def _gather_rows(table, idx):
    V, D = table.shape
    E = idx.shape[0]
    assert D % 16 == 0
    chunk = min(128, (120000 // (D + 1)) // 8 * 8)
    steps = -(-E // (_NW * chunk))
    E_pad = _NW * chunk * steps
    if E_pad != E:
        idx = jnp.concatenate([idx, jnp.zeros((E_pad - E,), jnp.int32)])
    out = _sc_gather_call(V, D, E_pad, chunk, steps)(table, idx)
    return out[:E] if E_pad != E else out


def _knn_body(posq_ref, post_ref, out_ref, *, self_mask, tq, p):
    j = pl.program_id(1)
    qx = posq_ref[0, :, 0:1]
    qy = posq_ref[0, :, 1:2]
    qz = posq_ref[0, :, 2:3]
    sx = post_ref[0, 0:1, :]
    sy = post_ref[0, 1:2, :]
    sz = post_ref[0, 2:3, :]
    dx = qx - sx
    dy = qy - sy
    dz = qz - sz
    d = dx * dx + dy * dy + dz * dz
    lane = lax.broadcasted_iota(jnp.int32, (tq, p), 1)
    if self_mask:
        row = j * tq + lax.broadcasted_iota(jnp.int32, (tq, 1), 0)
        d = jnp.where(lane == row, d + 1e10, d)
    for k in range(KNN):
        m = jnp.min(d, axis=1, keepdims=True)
        cand = jnp.where(d == m, lane, p)
        idx = jnp.min(cand, axis=1, keepdims=True)
        out_ref[0, :, k : k + 1] = idx
        d = jnp.where(lane == idx, jnp.float32(3e38), d)


def _knn(pos_q, pos_src_t, self_mask):
    B, Pq, _ = pos_q.shape
    P = pos_src_t.shape[2]
    tq = min(256, Pq)
    grid = (B, Pq // tq)
    return pl.pallas_call(
        functools.partial(_knn_body, self_mask=self_mask, tq=tq, p=P),
        grid=grid,
        in_specs=[
            pl.BlockSpec((1, tq, PPAD), lambda b, j: (b, j, 0)),
            pl.BlockSpec((1, 8, P), lambda b, j: (b, 0, 0)),
        ],
        out_specs=pl.BlockSpec((1, tq, KNN), lambda b, j: (b, j, 0)),
        out_shape=jax.ShapeDtypeStruct((B, Pq, KNN), jnp.int32),
    )(pos_q, pos_src_t)


def _fps_body(psq_ref, out_ref, *, n, p, rows, nb):
    lane = (lax.broadcasted_iota(jnp.int32, (rows, 128), 0) * 128
            + lax.broadcasted_iota(jnp.int32, (rows, 128), 1))
    coords = []
    for b in range(nb):
        coords.append((psq_ref[b, 0:rows, :],
                       psq_ref[b, rows : 2 * rows, :],
                       psq_ref[b, 2 * rows :, :]))

    def red2(x, op):
        return op(op(x, axis=0, keepdims=True), axis=1, keepdims=True)

    def dist(b, sx, sy, sz):
        X, Y, Z = coords[b]
        dx = X - sx
        dy = Y - sy
        dz = Z - sz
        return dx * dx + dy * dy + dz * dz

    def extract(b, oh):
        X, Y, Z = coords[b]
        return (red2(jnp.where(oh, X, 0.0), jnp.sum),
                red2(jnp.where(oh, Y, 0.0), jnp.sum),
                red2(jnp.where(oh, Z, 0.0), jnp.sum))

    carry0 = []
    oh0 = lane == 0
    for b in range(nb):
        sx, sy, sz = extract(b, oh0)
        mind = dist(b, sx, sy, sz)
        mind = jnp.where(lane < p, mind, -1.0)
        out_ref[b, 0, 0] = jnp.int32(0)
        carry0.append(mind)
        carry0.append(jnp.zeros((1, 1), jnp.int32))

    def step(i, carry):
        nxts = []
        for b in range(nb):
            mind, prev = carry[2 * b], carry[2 * b + 1]
            sx, sy, sz = extract(b, lane == prev)
            mind = jnp.minimum(mind, dist(b, sx, sy, sz))
            m = red2(mind, jnp.max)
            nxt = red2(jnp.where(mind == m, lane, p), jnp.min)
            out_ref[b, 0, i] = nxt[0, 0]
            nxts.append((mind, nxt))
        return tuple(x for mn in nxts for x in mn)

    lax.fori_loop(1, n, step, tuple(carry0))


def _fps(pos_sq, n, p):
    B, r3, _ = pos_sq.shape
    rows = r3 // 3
    out = pl.pallas_call(
        functools.partial(_fps_body, n=n, p=p, rows=rows, nb=B),
        in_specs=[pl.BlockSpec((B, r3, 128), lambda: (0, 0, 0))],
        out_specs=pl.BlockSpec(
            (B, 1, n), lambda: (0, 0, 0), memory_space=pltpu.SMEM
        ),
        out_shape=jax.ShapeDtypeStruct((B, 1, n), jnp.int32),
    )(pos_sq)
    return out[:, 0, :]


def _dense_body(x_ref, w_ref, b_ref, o_ref, *, scale):
    y = jnp.dot(x_ref[0], w_ref[...], preferred_element_type=jnp.float32)
    y = scale * (y + b_ref[...][None, :])
    o_ref[0] = jnp.maximum(y, 0.0)


def _dense(x, w, b, scale, tp):
    B, P, Cin = x.shape
    Cout = w.shape[1]
    grid = (B, P // tp)
    return pl.pallas_call(
        functools.partial(_dense_body, scale=scale),
        grid=grid,
        in_specs=[
            pl.BlockSpec((1, tp, Cin), lambda b, j: (b, j, 0)),
            pl.BlockSpec((Cin, Cout), lambda b, j: (0, 0)),
            pl.BlockSpec((Cout,), lambda b, j: (0,)),
        ],
        out_specs=pl.BlockSpec((1, tp, Cout), lambda b, j: (b, j, 0)),
        out_shape=jax.ShapeDtypeStruct((B, P, Cout), jnp.float32),
    )(x, w, b)


def _latent_body(x_ref, w_ref, b_ref, o_ref, *, scale):
    y = jnp.dot(x_ref[...], w_ref[...], preferred_element_type=jnp.float32)
    y = scale * (y + b_ref[...][None, :])
    o_ref[...] = jnp.maximum(y, 0.0)


def _latent(x, w, b, scale, tc):
    R, Cin = x.shape
    Cout = w.shape[1]
    grid = (Cout // tc,)
    return pl.pallas_call(
        functools.partial(_latent_body, scale=scale),
        grid=grid,
        in_specs=[
            pl.BlockSpec((R, Cin), lambda j: (0, 0)),
            pl.BlockSpec((Cin, tc), lambda j: (0, j)),
            pl.BlockSpec((tc,), lambda j: (j,)),
        ],
        out_specs=pl.BlockSpec((R, tc), lambda j: (0, j)),
        out_shape=jax.ShapeDtypeStruct((R, Cout), jnp.float32),
    )(x, w, b)


def _pre_body(g_ref, pos_ref, wi_ref, bi_ref, wl_ref, ws_ref, wd_ref,
              vsp_ref, q_ref, *, c):
    x = g_ref[0, 0]
    for k in range(1, KNN):
        x = jnp.maximum(x, g_ref[k, 0])
    h = jnp.dot(x, wi_ref[...], preferred_element_type=jnp.float32)
    h = jnp.maximum(h + bi_ref[...][None, :], 0.0)
    vsp_ref[0, :, 0:c] = jnp.dot(h, wl_ref[...], preferred_element_type=jnp.float32)
    vsp_ref[0, :, c : 2 * c] = jnp.dot(h, ws_ref[...], preferred_element_type=jnp.float32)
    vsp_ref[0, :, 2 * c : 2 * c + PPAD] = pos_ref[0]
    q_ref[0] = jnp.dot(h, wd_ref[...], preferred_element_type=jnp.float32)


def _pre(g, pos, tb, tp):
    _, B, P, C = g.shape
    D = 2 * C + PPAD
    grid = (B, P // tp)
    return pl.pallas_call(
        functools.partial(_pre_body, c=C),
        grid=grid,
        in_specs=[
            pl.BlockSpec((KNN, 1, tp, C), lambda b, j: (0, b, j, 0)),
            pl.BlockSpec((1, tp, PPAD), lambda b, j: (b, j, 0)),
            pl.BlockSpec((C, C), lambda b, j: (0, 0)),
            pl.BlockSpec((C,), lambda b, j: (0,)),
            pl.BlockSpec((C, C), lambda b, j: (0, 0)),
            pl.BlockSpec((C, C), lambda b, j: (0, 0)),
            pl.BlockSpec((C, C), lambda b, j: (0, 0)),
        ],
        out_specs=[
            pl.BlockSpec((1, tp, D), lambda b, j: (b, j, 0)),
            pl.BlockSpec((1, tp, C), lambda b, j: (b, j, 0)),
        ],
        out_shape=[
            jax.ShapeDtypeStruct((B, P, D), jnp.float32),
            jax.ShapeDtypeStruct((B, P, C), jnp.float32),
        ],
    )(g, pos, tb["lin_in"]["W"], tb["lin_in"]["b"], tb["lin"],
      tb["lin_src"], tb["lin_dst"])


def _mlp2_tile(x, w1, b1, w2, b2):
    t = jnp.dot(x, w1, preferred_element_type=jnp.float32)
    t = jnp.maximum(t + b1[None, :], 0.0)
    t = jnp.dot(t, w2, preferred_element_type=jnp.float32)
    return jnp.maximum(t + b2[None, :], 0.0)


def _edge_body(pos_ref, q_ref, g_ref, pw1_ref, pb1_ref, pw2_ref, pb2_ref,
               aw1_ref, ab1_ref, aw2_ref, ab2_ref, wo_ref, bo_ref, o_ref,
               *, c, tp):
    pos_i = pos_ref[0]
    q = q_ref[0]
    pw1 = pw1_ref[...]
    pb1 = pb1_ref[...]
    pw2 = pw2_ref[...]
    pb2 = pb2_ref[...]
    aw1 = aw1_ref[...]
    ab1 = ab1_ref[...]
    aw2 = aw2_ref[...]
    ab2 = ab2_ref[...]
    m = jnp.full((tp, c), -jnp.inf, jnp.float32)
    l = jnp.zeros((tp, c), jnp.float32)
    acc = jnp.zeros((tp, c), jnp.float32)
    for k in range(KNN + 1):
        gk = g_ref[k, 0]
        vk = gk[:, 0:c]
        sk = gk[:, c : 2 * c]
        pk = gk[:, 2 * c : 2 * c + PPAD]
        rel = pos_i - pk
        delta = _mlp2_tile(rel, pw1, pb1, pw2, pb2)
        alpha = _mlp2_tile(q - sk + delta, aw1, ab1, aw2, ab2)
        mk = jnp.maximum(m, alpha)
        corr = jnp.exp(m - mk)
        pexp = jnp.exp(alpha - mk)
        acc = acc * corr + pexp * (vk + delta)
        l = l * corr + pexp
        m = mk
    out = acc / l
    out = jnp.dot(out, wo_ref[...], preferred_element_type=jnp.float32)
    o_ref[0] = jnp.maximum(out + bo_ref[...][None, :], 0.0)


def _edge(pos, q, g, tb, pw1p, tp):
    _, B, P, D = g.shape
    C = (D - PPAD) // 2
    grid = (B, P // tp)
    return pl.pallas_call(
        functools.partial(_edge_body, c=C, tp=tp),
        grid=grid,
        in_specs=[
            pl.BlockSpec((1, tp, PPAD), lambda b, j: (b, j, 0)),
            pl.BlockSpec((1, tp, C), lambda b, j: (b, j, 0)),
            pl.BlockSpec((KNN + 1, 1, tp, D), lambda b, j: (0, b, j, 0)),
            pl.BlockSpec((PPAD, 64), lambda b, j: (0, 0)),
            pl.BlockSpec((64,), lambda b, j: (0,)),
            pl.BlockSpec((64, C), lambda b, j: (0, 0)),
            pl.BlockSpec((C,), lambda b, j: (0,)),
            pl.BlockSpec((C, 64), lambda b, j: (0, 0)),
            pl.BlockSpec((64,), lambda b, j: (0,)),
            pl.BlockSpec((64, C), lambda b, j: (0, 0)),
            pl.BlockSpec((C,), lambda b, j: (0,)),
            pl.BlockSpec((C, C), lambda b, j: (0, 0)),
            pl.BlockSpec((C,), lambda b, j: (0,)),
        ],
        out_specs=pl.BlockSpec((1, tp, C), lambda b, j: (b, j, 0)),
        out_shape=jax.ShapeDtypeStruct((B, P, C), jnp.float32),
    )(pos, q, g, pw1p, tb["pos_nn"][0]["b"], tb["pos_nn"][1]["W"],
      tb["pos_nn"][1]["b"], tb["attn_nn"][0]["W"], tb["attn_nn"][0]["b"],
      tb["attn_nn"][1]["W"], tb["attn_nn"][1]["b"], tb["lin_out"]["W"],
      tb["lin_out"]["b"])


def _edge0_body(pos_ref, g_ref, w1_ref, b1_ref, wi_ref, bi_ref, wl_ref,
                ws_ref, wd_ref, pw1_ref, pb1_ref, pw2_ref, pb2_ref,
                aw1_ref, ab1_ref, aw2_ref, ab2_ref, wo_ref, bo_ref, o_ref,
                *, c, tp):
    x0 = jnp.maximum(BN_SC * (w1_ref[...] + b1_ref[...][None, :]), 0.0)
    h = jnp.dot(x0, wi_ref[...], preferred_element_type=jnp.float32)
    h = jnp.maximum(h + bi_ref[...][None, :], 0.0)
    vc = jnp.dot(h, wl_ref[...], preferred_element_type=jnp.float32)
    sc = jnp.dot(h, ws_ref[...], preferred_element_type=jnp.float32)
    qc = jnp.dot(h, wd_ref[...], preferred_element_type=jnp.float32)
    qs = qc - sc
    pos_i = pos_ref[0]
    pw1 = pw1_ref[...]
    pb1 = pb1_ref[...]
    pw2 = pw2_ref[...]
    pb2 = pb2_ref[...]
    aw1 = aw1_ref[...]
    ab1 = ab1_ref[...]
    aw2 = aw2_ref[...]
    ab2 = ab2_ref[...]
    m = jnp.full((tp, c), -jnp.inf, jnp.float32)
    l = jnp.zeros((tp, c), jnp.float32)
    acc = jnp.zeros((tp, c), jnp.float32)
    for k in range(KNN + 1):
        pk = g_ref[k, 0]
        rel = pos_i - pk
        delta = _mlp2_tile(rel, pw1, pb1, pw2, pb2)
        alpha = _mlp2_tile(qs + delta, aw1, ab1, aw2, ab2)
        mk = jnp.maximum(m, alpha)
        corr = jnp.exp(m - mk)
        pexp = jnp.exp(alpha - mk)
        acc = acc * corr + pexp * (vc + delta)
        l = l * corr + pexp
        m = mk
    out = acc / l
    out = jnp.dot(out, wo_ref[...], preferred_element_type=jnp.float32)
    o_ref[0] = jnp.maximum(out + bo_ref[...][None, :], 0.0)


def _edge0(pos, g, mlp_in, tb, pw1p, tp):
    B, P, _ = pos.shape
    C = tb["lin"].shape[0]
    grid = (B, P // tp)
    return pl.pallas_call(
        functools.partial(_edge0_body, c=C, tp=tp),
        grid=grid,
        in_specs=[
            pl.BlockSpec((1, tp, PPAD), lambda b, j: (b, j, 0)),
            pl.BlockSpec((KNN + 1, 1, tp, PPAD), lambda b, j: (0, b, j, 0)),
            pl.BlockSpec((1, C), lambda b, j: (0, 0)),
            pl.BlockSpec((C,), lambda b, j: (0,)),
            pl.BlockSpec((C, C), lambda b, j: (0, 0)),
            pl.BlockSpec((C,), lambda b, j: (0,)),
            pl.BlockSpec((C, C), lambda b, j: (0, 0)),
            pl.BlockSpec((C, C), lambda b, j: (0, 0)),
            pl.BlockSpec((C, C), lambda b, j: (0, 0)),
            pl.BlockSpec((PPAD, 64), lambda b, j: (0, 0)),
            pl.BlockSpec((64,), lambda b, j: (0,)),
            pl.BlockSpec((64, C), lambda b, j: (0, 0)),
            pl.BlockSpec((C,), lambda b, j: (0,)),
            pl.BlockSpec((C, 64), lambda b, j: (0, 0)),
            pl.BlockSpec((64,), lambda b, j: (0,)),
            pl.BlockSpec((64, C), lambda b, j: (0, 0)),
            pl.BlockSpec((C,), lambda b, j: (0,)),
            pl.BlockSpec((C, C), lambda b, j: (0, 0)),
            pl.BlockSpec((C,), lambda b, j: (0,)),
        ],
        out_specs=pl.BlockSpec((1, tp, C), lambda b, j: (b, j, 0)),
        out_shape=jax.ShapeDtypeStruct((B, P, C), jnp.float32),
    )(pos, g, mlp_in["W"], mlp_in["b"], tb["lin_in"]["W"], tb["lin_in"]["b"],
      tb["lin"], tb["lin_src"], tb["lin_dst"], pw1p, tb["pos_nn"][0]["b"],
      tb["pos_nn"][1]["W"], tb["pos_nn"][1]["b"], tb["attn_nn"][0]["W"],
      tb["attn_nn"][0]["b"], tb["attn_nn"][1]["W"], tb["attn_nn"][1]["b"],
      tb["lin_out"]["W"], tb["lin_out"]["b"])


def _heads_body(f_ref, wo_ref, bo_ref, wc_ref, bc_ref, off_ref, cls_ref):
    f = f_ref[0]
    off = jnp.dot(f, wo_ref[...], preferred_element_type=jnp.float32)
    off_ref[0] = (off + bo_ref[...][None, :]) * POSR
    cl = jnp.dot(f, wc_ref[...], preferred_element_type=jnp.float32)
    cls_ref[0] = jax.nn.sigmoid(cl + bc_ref[...][None, :])


def _heads(feat, wo, bo, wc, bc, tp):
    B, P, C = feat.shape
    grid = (B, P // tp)
    return pl.pallas_call(
        _heads_body,
        grid=grid,
        in_specs=[
            pl.BlockSpec((1, tp, C), lambda b, j: (b, j, 0)),
            pl.BlockSpec((C, 3), lambda b, j: (0, 0)),
            pl.BlockSpec((3,), lambda b, j: (0,)),
            pl.BlockSpec((C, 1), lambda b, j: (0, 0)),
            pl.BlockSpec((1,), lambda b, j: (0,)),
        ],
        out_specs=[
            pl.BlockSpec((1, tp, 3), lambda b, j: (b, j, 0)),
            pl.BlockSpec((1, tp, 1), lambda b, j: (b, j, 0)),
        ],
        out_shape=[
            jax.ShapeDtypeStruct((B, P, 3), jnp.float32),
            jax.ShapeDtypeStruct((B, P, 1), jnp.float32),
        ],
    )(feat, wo, bo, wc, bc)


def _pad16(w3):
    pad = [(0, 0)] * w3.ndim
    for ax, sz in enumerate(w3.shape):
        if sz == 3:
            pad[ax] = (0, PPAD - 3)
    return jnp.pad(w3, pad)


def _edge_gather(vsp, nbr, B, P):
    D = vsp.shape[-1]
    self_idx = jnp.broadcast_to(
        jnp.arange(P, dtype=nbr.dtype)[None, :, None], (B, P, 1))
    nb = jnp.concatenate([nbr, self_idx], axis=2)
    boff = (jnp.arange(B, dtype=jnp.int32) * P)[:, None, None]
    flat = jnp.transpose(nb + boff, (2, 0, 1)).reshape(-1)
    rows = _gather_rows(vsp.reshape(B * P, D), flat)
    return rows.reshape(KNN + 1, B, P, D)


def _assign_gather(h, assign, B, P, Psub):
    C = h.shape[-1]
    boff = (jnp.arange(B, dtype=jnp.int32) * P)[:, None, None]
    flat = jnp.transpose(assign + boff, (2, 0, 1)).reshape(-1)
    rows = _gather_rows(h.reshape(B * P, C), flat)
    return rows.reshape(KNN, B, Psub, C)


def _pos_t(pos16, B, P):
    return jnp.transpose(pos16[:, :, :8], (0, 2, 1))


def _pos_sq(pos16, B, P):
    rows = -(-P // 128)
    c = jnp.transpose(pos16[:, :, :3], (0, 2, 1))
    c = jnp.pad(c, ((0, 0), (0, 0), (0, rows * 128 - P)))
    return c.reshape(B, 3 * rows, 128)


def kernel(points, vectors, params):
    del vectors
    B, P0, _ = points.shape
    dims = [params["tb"][i]["lin"].shape[0] for i in range(5)]

    tp_lvl = [512, 256, 256, 64, 16]
    ps = [P0]
    for _ in range(4):
        ps.append(int(np.ceil(0.25 * ps[-1])))

    pos16 = _pad16(points)

    post = _pos_t(pos16, B, P0)
    nbr0 = _knn(pos16, post, self_mask=True)
    g0 = _edge_gather(pos16, nbr0, B, P0)
    tb0 = params["tb"][0]
    pw1p0 = _pad16(tb0["pos_nn"][0]["W"])
    x = _edge0(pos16, g0, params["mlp_input"], tb0, pw1p0, tp_lvl[0])

    pos = pos16
    for i in range(4):
        P, Psub, C2 = ps[i], ps[i + 1], dims[i + 1]
        post = _pos_t(pos, B, P)
        fidx = _fps(_pos_sq(pos, B, P), Psub, P)
        boff = (jnp.arange(B, dtype=jnp.int32) * P)[:, None]
        sub_pos = _gather_rows(
            pos.reshape(B * P, PPAD), (fidx + boff).reshape(-1)
        ).reshape(B, Psub, PPAD)
        assign = _knn(sub_pos, post, self_mask=False)
        subt = _pos_t(sub_pos, B, Psub)
        nbr = _knn(sub_pos, subt, self_mask=True)

        td = params["td"][i]
        h = _dense(x, td["W"], td["b"], BN_SC, tp_lvl[i])
        g = _assign_gather(h, assign, B, P, Psub)

        tb = params["tb"][i + 1]
        vsp, q = _pre(g, sub_pos, tb, tp_lvl[i + 1])
        gj = _edge_gather(vsp, nbr, B, Psub)
        pw1p = _pad16(tb["pos_nn"][0]["W"])
        x = _edge(sub_pos, q, gj, tb, pw1p, tp_lvl[i + 1])
        pos = sub_pos

    lat = _latent(x.reshape(B * ps[4], dims[4]), params["lin"]["W"],
                  params["lin"]["b"], BN_SC, 2048)
    feat = lat.reshape(B, -1, 128)
    off, cls = _heads(
        feat, params["offset_fc"]["W"], params["offset_fc"]["b"],
        params["cls_fc"]["W"], params["cls_fc"]["b"], 512)
    return feat, cls[..., 0], off
